# baseline (speedup 1.0000x reference)
import functools

import jax
import jax.numpy as jnp
from jax import lax
from jax.experimental import pallas as pl
from jax.experimental.pallas import tpu as pltpu
from jax.experimental.pallas import tpu_sc as plsc

B, NRX, NRA, NTX, NTA = 4, 4, 4, 1, 64
NOFDM, NSC = 14, 512
NUM_BEAMS = 16
K = NUM_BEAMS // NRX
NGRP = B * NRX * NRA * NTX * NOFDM
ROWS_IN = NGRP * NTA
ROWS_OUT = NGRP * NUM_BEAMS
SCALE = 0.25

NC, NS, L = 2, 16, 16
NW = NC * NS
GRP_PER_W = NGRP // NW
GRP_PER_B = NRX * NRA * NOFDM
NBUF = 3


def _power_topk_body(h0, h1, h2, h3, idx_ref):
    p = jnp.zeros((NTA,), jnp.float32)
    for h_ref in (h0, h1, h2, h3):
        x = h_ref[0, 0, 0, 0]
        p = p + jnp.sum(x * x, axis=(0, 2))
    p2 = p.reshape(1, NTA)
    iota = lax.broadcasted_iota(jnp.int32, (1, NTA), 1)
    kiota = lax.broadcasted_iota(jnp.int32, (1, K), 1)
    idx_out = jnp.zeros((1, K), jnp.int32)
    for k in range(K):
        mx = jnp.max(p2)
        j = jnp.min(jnp.where(p2 == mx, iota, NTA))
        idx_out = jnp.where(kiota == k, j, idx_out)
        p2 = jnp.where(iota == j, -1.0, p2)
    idx_ref[0, 0] = idx_out


def _power_topk(hT, nb=B):
    return pl.pallas_call(
        _power_topk_body,
        grid=(nb, NRX),
        in_specs=[pl.BlockSpec((1, 1, 1, 1, NOFDM, NTA, NSC),
                               functools.partial(
                                   lambda A, b, r: (b, r, A, 0, 0, 0, 0), A))
                  for A in range(NRA)],
        out_specs=pl.BlockSpec((1, 1, 1, K), lambda b, r: (b, r, 0, 0)),
        out_shape=jax.ShapeDtypeStruct((nb, NRX, 1, K), jnp.int32),
    )(hT, hT, hT, hT)


@functools.cache
def _make_sc_gather(nbatch, base):
    ngrp_p = nbatch * GRP_PER_B
    gpw = ngrp_p // NW

    def body(table_hbm, idx_hbm, out_hbm, idx_v, trow_v, bufs, in_sems, out_sems):
        cid = lax.axis_index("c")
        sid = lax.axis_index("s")
        wid = sid * NC + cid
        pltpu.sync_copy(idx_hbm, idx_v)

        def issue(t):
            grp = wid * gpw + t
            b = grp // GRP_PER_B
            beams = idx_v[pl.ds(b * NUM_BEAMS, L)]
            m = t % NBUF
            trow_v[m, :] = grp * NTA + beams
            return pltpu.async_copy(
                table_hbm.at[trow_v.at[m]], bufs.at[m], in_sems.at[m])

        UNROLL = 8

        def scale(m):
            def srow(rr, carry):
                def scol(q, carry2):
                    for u in range(UNROLL):
                        sl = pl.ds((q * UNROLL + u) * L, L)
                        bufs[m, rr, sl] = bufs[m, rr, sl] * SCALE
                    return carry2
                return lax.fori_loop(0, NSC // (L * UNROLL), scol, carry)
            lax.fori_loop(0, NUM_BEAMS, srow, 0)

        in_cp = {0: issue(0)}
        out_cp = {}
        for t in range(gpw):
            m = t % NBUF
            if t + 1 < gpw:
                if t + 1 >= NBUF:
                    out_cp[(t + 1) % NBUF].wait()
                in_cp[t + 1] = issue(t + 1)
            in_cp[t].wait()
            scale(m)
            grp = wid * gpw + t
            orow = (base * GRP_PER_B + grp) * NUM_BEAMS
            out_cp[m] = pltpu.async_copy(
                bufs.at[m], out_hbm.at[pl.ds(orow, NUM_BEAMS)], out_sems.at[m])
        for m in range(min(NBUF, gpw)):
            out_cp[m].wait()

    return pl.kernel(
        body,
        out_type=(),
        mesh=plsc.VectorSubcoreMesh(core_axis_name="c", subcore_axis_name="s"),
        scratch_types=[
            pltpu.VMEM((nbatch * NUM_BEAMS,), jnp.int32),
            pltpu.VMEM((NBUF, NUM_BEAMS), jnp.int32),
            pltpu.VMEM((NBUF, NUM_BEAMS, NSC), jnp.float32),
            pltpu.SemaphoreType.DMA((NBUF,)),
            pltpu.SemaphoreType.DMA((NBUF,)),
        ],
    )


HB = 2


def kernel(h_channel):
    hT = jnp.transpose(h_channel, (0, 1, 2, 3, 5, 4, 6))
    out_ref = jax.empty_ref(jax.ShapeDtypeStruct((ROWS_OUT, NSC), jnp.float32))
    for half in range(B // HB):
        hh = lax.slice_in_dim(hT, half * HB, (half + 1) * HB, axis=0)
        idx_h = _power_topk(hh, HB).reshape(HB * NUM_BEAMS)
        table_h = hh.reshape(HB * GRP_PER_B * NTA, NSC)
        _make_sc_gather(HB, half * HB)(table_h, idx_h, out_ref)
    out = out_ref[...].reshape(B, NRX, NRA, NTX, NOFDM, NUM_BEAMS, NSC)
    return jnp.transpose(out, (0, 1, 2, 3, 5, 4, 6))


# speedup vs baseline: 1.8044x; 1.8044x over previous
import functools

import jax
import jax.numpy as jnp
from jax import lax
from jax.experimental import pallas as pl
from jax.experimental.pallas import tpu as pltpu
from jax.experimental.pallas import tpu_sc as plsc

B, NRX, NRA, NTX, NTA = 4, 4, 4, 1, 64
NOFDM, NSC = 14, 512
NUM_BEAMS = 16
K = NUM_BEAMS // NRX
NGRP = B * NRX * NRA * NTX * NOFDM
ROWS_IN = NGRP * NTA
ROWS_OUT = NGRP * NUM_BEAMS
SCALE = 0.25

NC, NS, L = 2, 16, 16
NW = NC * NS
GRP_PER_W = NGRP // NW
GRP_PER_B = NRX * NRA * NOFDM
NBUF = 3


def _power_topk_body(h0, h1, h2, h3, idx_ref):
    p = jnp.zeros((NTA,), jnp.float32)
    for h_ref in (h0, h1, h2, h3):
        x = h_ref[0, 0, 0, 0]
        p = p + jnp.sum(x * x, axis=(0, 2))
    p2 = p.reshape(1, NTA)
    iota = lax.broadcasted_iota(jnp.int32, (1, NTA), 1)
    kiota = lax.broadcasted_iota(jnp.int32, (1, K), 1)
    idx_out = jnp.zeros((1, K), jnp.int32)
    for k in range(K):
        mx = jnp.max(p2)
        j = jnp.min(jnp.where(p2 == mx, iota, NTA))
        idx_out = jnp.where(kiota == k, j, idx_out)
        p2 = jnp.where(iota == j, -1.0, p2)
    idx_ref[0, 0] = idx_out


def _power_topk(hT):
    return pl.pallas_call(
        _power_topk_body,
        grid=(B, NRX),
        in_specs=[pl.BlockSpec((1, 1, 1, 1, NOFDM, NTA, NSC),
                               functools.partial(
                                   lambda A, b, r: (b, r, A, 0, 0, 0, 0), A))
                  for A in range(NRA)],
        out_specs=pl.BlockSpec((1, 1, 1, K), lambda b, r: (b, r, 0, 0)),
        out_shape=jax.ShapeDtypeStruct((B, NRX, 1, K), jnp.int32),
    )(hT, hT, hT, hT)


@functools.cache
def _make_sc_gather():
    def body(table_hbm, idx_hbm, out_hbm, idx_v, trow_v, bufs, in_sems, out_sems):
        cid = lax.axis_index("c")
        sid = lax.axis_index("s")
        wid = sid * NC + cid
        pltpu.sync_copy(idx_hbm, idx_v)

        def issue(t):
            grp = wid * GRP_PER_W + t
            b = grp // GRP_PER_B
            beams = idx_v[pl.ds(b * NUM_BEAMS, L)]
            m = t % NBUF
            trow_v[m, :] = grp * NTA + beams
            return pltpu.async_copy(
                table_hbm.at[trow_v.at[m]], bufs.at[m], in_sems.at[m])

        UNROLL = 8

        def scale(m):
            def srow(rr, carry):
                def scol(q, carry2):
                    for u in range(UNROLL):
                        sl = pl.ds((q * UNROLL + u) * L, L)
                        bufs[m, rr, sl] = bufs[m, rr, sl] * SCALE
                    return carry2
                return lax.fori_loop(0, NSC // (L * UNROLL), scol, carry)
            lax.fori_loop(0, NUM_BEAMS, srow, 0)

        in_cp = {0: issue(0)}
        out_cp = {}
        for t in range(GRP_PER_W):
            m = t % NBUF
            if t + 1 < GRP_PER_W:
                if t + 1 >= NBUF:
                    out_cp[(t + 1) % NBUF].wait()
                in_cp[t + 1] = issue(t + 1)
            in_cp[t].wait()
            scale(m)
            grp = wid * GRP_PER_W + t
            out_cp[m] = pltpu.async_copy(
                bufs.at[m], out_hbm.at[pl.ds(grp * NUM_BEAMS, NUM_BEAMS)],
                out_sems.at[m])
        for m in range(min(NBUF, GRP_PER_W)):
            out_cp[m].wait()

    return pl.kernel(
        body,
        out_type=jax.ShapeDtypeStruct((ROWS_OUT, NSC), jnp.float32),
        mesh=plsc.VectorSubcoreMesh(core_axis_name="c", subcore_axis_name="s"),
        scratch_types=[
            pltpu.VMEM((B * NUM_BEAMS,), jnp.int32),
            pltpu.VMEM((NBUF, NUM_BEAMS), jnp.int32),
            pltpu.VMEM((NBUF, NUM_BEAMS, NSC), jnp.float32),
            pltpu.SemaphoreType.DMA((NBUF,)),
            pltpu.SemaphoreType.DMA((NBUF,)),
        ],
    )


def kernel(h_channel):
    hT = jnp.transpose(h_channel, (0, 1, 2, 3, 5, 4, 6))
    idx = _power_topk(hT)
    idx_flat = idx.reshape(B * NUM_BEAMS)
    table = hT.reshape(ROWS_IN, NSC)
    outT = _make_sc_gather()(table, idx_flat)
    out = outT.reshape(B, NRX, NRA, NTX, NOFDM, NUM_BEAMS, NSC)
    return jnp.transpose(out, (0, 1, 2, 3, 5, 4, 6))
